# baseline (speedup 1.0000x reference)
import jax
import jax.numpy as jnp
from jax import lax
from jax.experimental import pallas as pl
from jax.experimental.pallas import tpu as pltpu

HIDDEN = 768
EDGE_FEAT = 9
MAX_PANELS = 37
MAX_EDGES = 39
NUM_STITCH = 8
CAT = 96
NBUF = 4


def _body(ep_ref, pidx_ref, eidx_ref, sidx_ref, w_ref, b_ref, tab_ref,
          g_ref, beta_ref, out_ref, scratch_ref, sem_ref):
    i = pl.program_id(0)
    n = pl.num_programs(0)
    slot = lax.rem(i, NBUF)

    @pl.when(i >= NBUF)
    def _():
        pltpu.make_async_copy(scratch_ref.at[slot], out_ref.at[i - NBUF],
                              sem_ref.at[slot]).wait()

    T = ep_ref.shape[1]
    ep = ep_ref[0]
    acc = jnp.dot(ep, w_ref[...], preferred_element_type=jnp.float32)
    acc = acc + b_ref[...]

    p = pidx_ref[0]
    e = eidx_ref[0] + MAX_PANELS
    s = sidx_ref[0] + (MAX_PANELS + MAX_EDGES)
    cols = lax.broadcasted_iota(jnp.int32, (T, CAT), 1)
    oh = ((cols == p).astype(jnp.float32)
          + (cols == e).astype(jnp.float32)
          + (cols == s).astype(jnp.float32))
    acc = acc + jnp.dot(oh, tab_ref[...], preferred_element_type=jnp.float32)

    mean = jnp.mean(acc, axis=1, keepdims=True)
    cen = acc - mean
    var = jnp.mean(cen * cen, axis=1, keepdims=True)
    inv = lax.rsqrt(var + 1e-5)
    scratch_ref[slot] = cen * inv * g_ref[...] + beta_ref[...]

    pltpu.make_async_copy(scratch_ref.at[slot], out_ref.at[i],
                          sem_ref.at[slot]).start()

    @pl.when(i == n - 1)
    def _():
        for k in range(NBUF):
            step = n - NBUF + k
            pltpu.make_async_copy(scratch_ref.at[lax.rem(step, NBUF)],
                                  out_ref.at[step],
                                  sem_ref.at[lax.rem(step, NBUF)]).wait()


def kernel(edge_parameters, stitch_types, panel_indices, edge_indices,
           W_edge, b_edge, panel_tab, edge_tab, stitch_tab, ln_gamma, ln_beta):
    B, P, E, F = edge_parameters.shape
    T = P * E
    ep = edge_parameters.reshape(B, T, F)
    pidx = panel_indices.reshape(B, T, 1).astype(jnp.int32)
    eidx = edge_indices.reshape(B, T, 1).astype(jnp.int32)
    sidx = stitch_types.reshape(B, T, 1).astype(jnp.int32)
    tab = jnp.concatenate(
        [panel_tab, edge_tab, stitch_tab,
         jnp.zeros((CAT - MAX_PANELS - MAX_EDGES - NUM_STITCH, HIDDEN),
                   jnp.float32)], axis=0)

    tok_spec = pl.BlockSpec((1, T, EDGE_FEAT), lambda i: (i, 0, 0))
    idx_spec = pl.BlockSpec((1, T, 1), lambda i: (i, 0, 0))
    full = lambda shape: pl.BlockSpec(shape, lambda i: (0,) * len(shape))
    out = pl.pallas_call(
        _body,
        grid=(B,),
        in_specs=[
            tok_spec, idx_spec, idx_spec, idx_spec,
            full((EDGE_FEAT, HIDDEN)),
            full((1, HIDDEN)),
            full((CAT, HIDDEN)),
            full((1, HIDDEN)),
            full((1, HIDDEN)),
        ],
        out_specs=pl.BlockSpec(memory_space=pltpu.MemorySpace.HBM),
        out_shape=jax.ShapeDtypeStruct((B, T, HIDDEN), jnp.float32),
        scratch_shapes=[
            pltpu.VMEM((NBUF, T, HIDDEN), jnp.float32),
            pltpu.SemaphoreType.DMA((NBUF,)),
        ],
    )(ep, pidx, eidx, sidx, W_edge, b_edge.reshape(1, HIDDEN), tab,
      ln_gamma.reshape(1, HIDDEN), ln_beta.reshape(1, HIDDEN))
    return out
